# Initial kernel scaffold; baseline (speedup 1.0000x reference)
import jax
import jax.numpy as jnp
from jax.experimental import pallas as pl


def kernel(x, edge_index, batch, W_h1, b_h1, W_h2, b_h2, W_enc, b_enc, ln_g, ln_b, W1, b1, W2, b2, W_dec, b_dec):
    raise NotImplementedError("write your pallas kernel here")


# speedup vs baseline: 6.7609x; 6.7609x over previous
import functools

import jax
import jax.numpy as jnp
from jax import lax
from jax.experimental import pallas as pl
from jax.experimental.pallas import tpu as pltpu
from jax.experimental.pallas import tpu_sc as plsc

N = 50000
E = 800000
B = 5
IN_DIM = 10000
D = 100
L = 3
EPS = 1e-7

NP = 50176
EP = 851968
C = 8
K = 13
KC0 = 7
RW = 2 * C
NB = 1024
NBLK = NP // NB
WBATCH = 4096
SROWS = EP // 128
ITERS = 13
ROWS_TILE = 416
ACC_TILE = NP // 16

_f32 = jnp.float32


def _head_body(x_ref, w1_ref, b1_ref, w2_ref, b2_ref, o_ref):
    h1 = jnp.dot(x_ref[...], w1_ref[...], preferred_element_type=_f32) + b1_ref[...]
    h1 = jnp.where(h1 > 0, h1, 0.01 * h1)
    h2 = jnp.dot(h1, w2_ref[...], preferred_element_type=_f32) + b2_ref[...]
    o_ref[...] = jnp.where(h2 > 0, h2, 0.01 * h2)


def _head(x2d, W_h1, b_h1, W_h2, b_h2):
    return pl.pallas_call(
        _head_body,
        out_shape=jax.ShapeDtypeStruct((B, D), _f32),
    )(x2d, W_h1, b_h1, W_h2, b_h2)


def _enc_body(x_ref, w_ref, b_ref, o_ref):
    o_ref[...] = x_ref[...] * w_ref[...] + b_ref[...]


def _enc(xp, W_enc, b_enc):
    return pl.pallas_call(
        _enc_body,
        grid=(NBLK,),
        in_specs=[
            pl.BlockSpec((NB, 1), lambda i: (i, 0)),
            pl.BlockSpec((1, D), lambda i: (0, 0)),
            pl.BlockSpec((1, D), lambda i: (0, 0)),
        ],
        out_specs=pl.BlockSpec((NB, D), lambda i: (i, 0)),
        out_shape=jax.ShapeDtypeStruct((NP, D), _f32),
    )(xp, W_enc, b_enc)


def _pre_body(h_ref, g_ref, b_ref, t_ref, u_ref):
    h = h_ref[...]
    mu = jnp.mean(h, axis=1, keepdims=True)
    var = jnp.mean((h - mu) * (h - mu), axis=1, keepdims=True)
    t = (h - mu) * lax.rsqrt(var + 1e-5) * g_ref[...] + b_ref[...]
    t = jnp.maximum(t, 0.0)
    t_ref[...] = t
    m = t + EPS
    p = jnp.exp(m)
    mp = m * p
    z4 = jnp.zeros((NB, 4), _f32)
    p = jnp.concatenate([p, z4], axis=1)
    mp = jnp.concatenate([mp, z4], axis=1)
    for k in range(K):
        u_ref[k] = jnp.concatenate(
            [p[:, k * C:(k + 1) * C], mp[:, k * C:(k + 1) * C]], axis=1)


def _pre(h, g, b):
    return pl.pallas_call(
        _pre_body,
        grid=(NBLK,),
        in_specs=[
            pl.BlockSpec((NB, D), lambda i: (i, 0)),
            pl.BlockSpec((1, D), lambda i: (0, 0)),
            pl.BlockSpec((1, D), lambda i: (0, 0)),
        ],
        out_specs=[
            pl.BlockSpec((NB, D), lambda i: (i, 0)),
            pl.BlockSpec((K, NB, RW), lambda i: (0, i, 0)),
        ],
        out_shape=[
            jax.ShapeDtypeStruct((NP, D), _f32),
            jax.ShapeDtypeStruct((K, NP, RW), _f32),
        ],
    )(h, g, b)


def _post_body(sw_ref, t_ref, h_ref, w1_ref, b1_ref, w2_ref, b2_ref, o_ref):
    parts = [sw_ref[k] for k in range(K)]
    s = jnp.concatenate([q[:, :C] for q in parts], axis=1)[:, :D]
    w = jnp.concatenate([q[:, C:] for q in parts], axis=1)[:, :D]
    agg = w / (s + 1e-16)
    out = agg + t_ref[...]
    hid = jnp.dot(out, w1_ref[...], preferred_element_type=_f32) + b1_ref[...]
    hid = jnp.maximum(hid, 0.0)
    o_ref[...] = h_ref[...] + jnp.dot(hid, w2_ref[...],
                                      preferred_element_type=_f32) + b2_ref[...]


def _post(sw, t, h, W1, b1, W2, b2):
    return pl.pallas_call(
        _post_body,
        grid=(NBLK,),
        in_specs=[
            pl.BlockSpec((K, NB, RW), lambda i: (0, i, 0)),
            pl.BlockSpec((NB, D), lambda i: (i, 0)),
            pl.BlockSpec((NB, D), lambda i: (i, 0)),
            pl.BlockSpec((D, 2 * D), lambda i: (0, 0)),
            pl.BlockSpec((1, 2 * D), lambda i: (0, 0)),
            pl.BlockSpec((2 * D, D), lambda i: (0, 0)),
            pl.BlockSpec((1, D), lambda i: (0, 0)),
        ],
        out_specs=pl.BlockSpec((NB, D), lambda i: (i, 0)),
        out_shape=jax.ShapeDtypeStruct((NP, D), _f32),
    )(sw, t, h, W1, b1, W2, b2)


def _final_body(h_ref, oh_ref, xs_ref, wd_ref, bd_ref, o_ref, acc_ref, cnt_ref):
    i = pl.program_id(0)

    @pl.when(i == 0)
    def _():
        acc_ref[...] = jnp.zeros_like(acc_ref)
        cnt_ref[...] = jnp.zeros_like(cnt_ref)

    oh = oh_ref[...]
    acc_ref[...] += lax.dot_general(oh, h_ref[...], (((0,), (0,)), ((), ())),
                                    preferred_element_type=_f32)
    cnt_ref[...] += lax.dot_general(oh, jnp.ones((NB, 1), _f32),
                                    (((0,), (0,)), ((), ())),
                                    preferred_element_type=_f32)

    @pl.when(i == NBLK - 1)
    def _():
        pooled = acc_ref[...] / jnp.maximum(cnt_ref[...], 1.0)
        o = 0.5 * xs_ref[...] + 0.5 * pooled
        o_ref[...] = jnp.dot(o, wd_ref[...], preferred_element_type=_f32) + bd_ref[...]


def _final(h, oh, xs, W_dec, b_dec):
    return pl.pallas_call(
        _final_body,
        grid=(NBLK,),
        in_specs=[
            pl.BlockSpec((NB, D), lambda i: (i, 0)),
            pl.BlockSpec((NB, B), lambda i: (i, 0)),
            pl.BlockSpec((B, D), lambda i: (0, 0)),
            pl.BlockSpec((D, 1), lambda i: (0, 0)),
            pl.BlockSpec((1, 1), lambda i: (0, 0)),
        ],
        out_specs=pl.BlockSpec((B, 1), lambda i: (0, 0)),
        out_shape=jax.ShapeDtypeStruct((B, 1), _f32),
        scratch_shapes=[
            pltpu.VMEM((B, D), _f32),
            pltpu.VMEM((B, 1), _f32),
        ],
    )(h, oh, xs, W_dec, b_dec)


_mesh = plsc.VectorSubcoreMesh(core_axis_name="c", subcore_axis_name="s")


@functools.partial(
    pl.kernel,
    out_type=jax.ShapeDtypeStruct((K, NP, RW), _f32),
    mesh=_mesh,
    scratch_types=[
        pltpu.VMEM((32, 128), jnp.int32),
        pltpu.VMEM((32, 128), jnp.int32),
        pltpu.VMEM((WBATCH, RW), _f32),
        pltpu.VMEM_SHARED((NP, RW), _f32),
        pltpu.SemaphoreType.DMA,
        pltpu.SemaphoreType.DMA,
    ],
    compiler_params=pltpu.CompilerParams(use_tc_tiling_on_sc=False),
)
def _sc_spmm(u_hbm, src_hbm, dst_hbm, z_hbm, out_hbm,
             sbuf, dbuf, gbuf, acc, gsem, ssem):
    c = lax.axis_index("c")
    s = lax.axis_index("s")

    for item in range(KC0):
        chunk = jnp.where(c == 0, item, KC0 + item)
        active = (c == 0) | (item < (K - KC0))

        @pl.when(active)
        def _():
            pltpu.sync_copy(z_hbm, acc.at[pl.ds(s * ACC_TILE, ACC_TILE)])
            plsc.subcore_barrier()

            def _body(g, carry):
                r = s * ROWS_TILE + g * 32
                pltpu.sync_copy(src_hbm.at[pl.ds(chunk * SROWS + r, 32)], sbuf)
                pltpu.sync_copy(dst_hbm.at[pl.ds(r, 32)], dbuf)
                gets = [
                    pltpu.async_copy(u_hbm.at[sbuf.at[j]],
                                     gbuf.at[pl.ds(j * 128, 128)], gsem)
                    for j in range(32)
                ]
                for dsc in gets:
                    dsc.wait()
                puts = [
                    pltpu.async_copy(gbuf.at[pl.ds(j * 128, 128)],
                                     acc.at[dbuf.at[j]], ssem, add=True)
                    for j in range(32)
                ]
                for dsc in puts:
                    dsc.wait()
                return carry

            lax.fori_loop(0, ITERS, _body, 0)
            plsc.subcore_barrier()
            pltpu.sync_copy(acc.at[pl.ds(s * ACC_TILE, ACC_TILE)],
                            out_hbm.at[chunk, pl.ds(s * ACC_TILE, ACC_TILE)])
            plsc.subcore_barrier()


def kernel(x, edge_index, batch, W_h1, b_h1, W_h2, b_h2, W_enc, b_enc,
           ln_g, ln_b, W1, b1, W2, b2, W_dec, b_dec):
    xs = _head(x.reshape(B, IN_DIM), W_h1, b_h1.reshape(1, -1),
               W_h2, b_h2.reshape(1, -1))

    xp = jnp.concatenate([x, jnp.zeros((NP - N, 1), _f32)], axis=0)
    pad_idx = (N + (jnp.arange(EP - E, dtype=jnp.int32) % (NP - N))).astype(jnp.int32)
    src = jnp.concatenate([edge_index[0], pad_idx])
    dst = jnp.concatenate([edge_index[1], pad_idx])
    srcK = (src[None, :] + (jnp.arange(K, dtype=jnp.int32) * NP)[:, None])
    srcK = srcK.reshape(K * SROWS, 128)
    dst2 = dst.reshape(SROWS, 128)
    zrows = jnp.zeros((ACC_TILE, RW), _f32)
    batch_pad = jnp.concatenate(
        [batch, jnp.full((NP - N,), B, dtype=batch.dtype)])
    oh = (batch_pad[:, None] == jnp.arange(B, dtype=batch.dtype)[None, :])
    oh = oh.astype(_f32)

    h = _enc(xp, W_enc.reshape(1, D), b_enc.reshape(1, D))
    for i in range(L):
        t, u = _pre(h, ln_g[i].reshape(1, D), ln_b[i].reshape(1, D))
        sw = _sc_spmm(u.reshape(K * NP, RW), srcK, dst2, zrows)
        h = _post(sw, t, h, W1[i], b1[i].reshape(1, -1),
                  W2[i], b2[i].reshape(1, -1))

    o = _final(h, oh, xs, W_dec, b_dec.reshape(1, 1))
    return o[:, 0]


# speedup vs baseline: 7.6399x; 1.1300x over previous
import functools

import jax
import jax.numpy as jnp
from jax import lax
from jax.experimental import pallas as pl
from jax.experimental.pallas import tpu as pltpu
from jax.experimental.pallas import tpu_sc as plsc

N = 50000
E = 800000
B = 5
IN_DIM = 10000
D = 100
L = 3
EPS = 1e-7

NP = 50176
EP = 851968
C = 8
K = 13
KC0 = 7
RW = 2 * C
NB = 1024
NBLK = NP // NB
WBATCH = 4096
SROWS = EP // 128
ITERS = 13
ROWS_TILE = 416
ACC_TILE = NP // 16

_f32 = jnp.float32


def _head_body(x_ref, w1_ref, b1_ref, w2_ref, b2_ref, o_ref):
    h1 = jnp.dot(x_ref[...], w1_ref[...], preferred_element_type=_f32) + b1_ref[...]
    h1 = jnp.where(h1 > 0, h1, 0.01 * h1)
    h2 = jnp.dot(h1, w2_ref[...], preferred_element_type=_f32) + b2_ref[...]
    o_ref[...] = jnp.where(h2 > 0, h2, 0.01 * h2)


def _head(x2d, W_h1, b_h1, W_h2, b_h2):
    return pl.pallas_call(
        _head_body,
        out_shape=jax.ShapeDtypeStruct((B, D), _f32),
    )(x2d, W_h1, b_h1, W_h2, b_h2)


def _enc_body(x_ref, w_ref, b_ref, o_ref):
    o_ref[...] = x_ref[...] * w_ref[...] + b_ref[...]


def _enc(xp, W_enc, b_enc):
    return pl.pallas_call(
        _enc_body,
        grid=(NBLK,),
        in_specs=[
            pl.BlockSpec((NB, 1), lambda i: (i, 0)),
            pl.BlockSpec((1, D), lambda i: (0, 0)),
            pl.BlockSpec((1, D), lambda i: (0, 0)),
        ],
        out_specs=pl.BlockSpec((NB, D), lambda i: (i, 0)),
        out_shape=jax.ShapeDtypeStruct((NP, D), _f32),
    )(xp, W_enc, b_enc)


def _pre_body(h_ref, g_ref, b_ref, t_ref, u_ref):
    h = h_ref[...]
    mu = jnp.mean(h, axis=1, keepdims=True)
    var = jnp.mean((h - mu) * (h - mu), axis=1, keepdims=True)
    t = (h - mu) * lax.rsqrt(var + 1e-5) * g_ref[...] + b_ref[...]
    t = jnp.maximum(t, 0.0)
    t_ref[...] = t
    m = t + EPS
    p = jnp.exp(m)
    mp = m * p
    z4 = jnp.zeros((NB, 4), _f32)
    p = jnp.concatenate([p, z4], axis=1)
    mp = jnp.concatenate([mp, z4], axis=1)
    for k in range(K):
        u_ref[k] = jnp.concatenate(
            [p[:, k * C:(k + 1) * C], mp[:, k * C:(k + 1) * C]], axis=1)


def _pre(h, g, b):
    return pl.pallas_call(
        _pre_body,
        grid=(NBLK,),
        in_specs=[
            pl.BlockSpec((NB, D), lambda i: (i, 0)),
            pl.BlockSpec((1, D), lambda i: (0, 0)),
            pl.BlockSpec((1, D), lambda i: (0, 0)),
        ],
        out_specs=[
            pl.BlockSpec((NB, D), lambda i: (i, 0)),
            pl.BlockSpec((K, NB, RW), lambda i: (0, i, 0)),
        ],
        out_shape=[
            jax.ShapeDtypeStruct((NP, D), _f32),
            jax.ShapeDtypeStruct((K, NP, RW), _f32),
        ],
    )(h, g, b)


def _post_body(sw_ref, t_ref, h_ref, w1_ref, b1_ref, w2_ref, b2_ref, o_ref):
    parts = [sw_ref[k] for k in range(K)]
    s = jnp.concatenate([q[:, :C] for q in parts], axis=1)[:, :D]
    w = jnp.concatenate([q[:, C:] for q in parts], axis=1)[:, :D]
    agg = w / (s + 1e-16)
    out = agg + t_ref[...]
    hid = jnp.dot(out, w1_ref[...], preferred_element_type=_f32) + b1_ref[...]
    hid = jnp.maximum(hid, 0.0)
    o_ref[...] = h_ref[...] + jnp.dot(hid, w2_ref[...],
                                      preferred_element_type=_f32) + b2_ref[...]


def _post(sw, t, h, W1, b1, W2, b2):
    return pl.pallas_call(
        _post_body,
        grid=(NBLK,),
        in_specs=[
            pl.BlockSpec((K, NB, RW), lambda i: (0, i, 0)),
            pl.BlockSpec((NB, D), lambda i: (i, 0)),
            pl.BlockSpec((NB, D), lambda i: (i, 0)),
            pl.BlockSpec((D, 2 * D), lambda i: (0, 0)),
            pl.BlockSpec((1, 2 * D), lambda i: (0, 0)),
            pl.BlockSpec((2 * D, D), lambda i: (0, 0)),
            pl.BlockSpec((1, D), lambda i: (0, 0)),
        ],
        out_specs=pl.BlockSpec((NB, D), lambda i: (i, 0)),
        out_shape=jax.ShapeDtypeStruct((NP, D), _f32),
    )(sw, t, h, W1, b1, W2, b2)


def _final_body(h_ref, oh_ref, xs_ref, wd_ref, bd_ref, o_ref, acc_ref, cnt_ref):
    i = pl.program_id(0)

    @pl.when(i == 0)
    def _():
        acc_ref[...] = jnp.zeros_like(acc_ref)
        cnt_ref[...] = jnp.zeros_like(cnt_ref)

    oh = oh_ref[...]
    acc_ref[...] += lax.dot_general(oh, h_ref[...], (((0,), (0,)), ((), ())),
                                    preferred_element_type=_f32)
    cnt_ref[...] += lax.dot_general(oh, jnp.ones((NB, 1), _f32),
                                    (((0,), (0,)), ((), ())),
                                    preferred_element_type=_f32)

    @pl.when(i == NBLK - 1)
    def _():
        pooled = acc_ref[...] / jnp.maximum(cnt_ref[...], 1.0)
        o = 0.5 * xs_ref[...] + 0.5 * pooled
        o_ref[...] = jnp.dot(o, wd_ref[...], preferred_element_type=_f32) + bd_ref[...]


def _final(h, oh, xs, W_dec, b_dec):
    return pl.pallas_call(
        _final_body,
        grid=(NBLK,),
        in_specs=[
            pl.BlockSpec((NB, D), lambda i: (i, 0)),
            pl.BlockSpec((NB, B), lambda i: (i, 0)),
            pl.BlockSpec((B, D), lambda i: (0, 0)),
            pl.BlockSpec((D, 1), lambda i: (0, 0)),
            pl.BlockSpec((1, 1), lambda i: (0, 0)),
        ],
        out_specs=pl.BlockSpec((B, 1), lambda i: (0, 0)),
        out_shape=jax.ShapeDtypeStruct((B, 1), _f32),
        scratch_shapes=[
            pltpu.VMEM((B, D), _f32),
            pltpu.VMEM((B, 1), _f32),
        ],
    )(h, oh, xs, W_dec, b_dec)


_mesh = plsc.VectorSubcoreMesh(core_axis_name="c", subcore_axis_name="s")


@functools.partial(
    pl.kernel,
    out_type=jax.ShapeDtypeStruct((K, NP, RW), _f32),
    mesh=_mesh,
    scratch_types=[
        pltpu.VMEM((32, 128), jnp.int32),
        pltpu.VMEM((32, 128), jnp.int32),
        pltpu.VMEM((WBATCH, RW), _f32),
        pltpu.VMEM_SHARED((NP, RW), _f32),
        pltpu.SemaphoreType.DMA,
        pltpu.SemaphoreType.DMA,
    ],
    compiler_params=pltpu.CompilerParams(use_tc_tiling_on_sc=False),
)
def _sc_spmm(u_hbm, src_hbm, dst_hbm, z_hbm, out_hbm,
             sbuf, dbuf, gbuf, acc, gsem, ssem):
    c = lax.axis_index("c")
    s = lax.axis_index("s")

    for item in range(KC0):
        chunk = jnp.where(c == 0, item, KC0 + item)
        active = (c == 0) | (item < (K - KC0))

        @pl.when(active)
        def _():
            pltpu.sync_copy(z_hbm, acc.at[pl.ds(s * ACC_TILE, ACC_TILE)])
            plsc.subcore_barrier()

            def _body(g, carry):
                r = s * ROWS_TILE + g * 32
                pltpu.sync_copy(src_hbm.at[pl.ds(chunk * SROWS + r, 32)], sbuf)
                pltpu.sync_copy(dst_hbm.at[pl.ds(r, 32)], dbuf)
                gets = [
                    pltpu.async_copy(u_hbm.at[sbuf.at[j]],
                                     gbuf.at[pl.ds(j * 128, 128)], gsem)
                    for j in range(32)
                ]
                puts = []
                for j in range(32):
                    gets[j].wait()
                    puts.append(
                        pltpu.async_copy(gbuf.at[pl.ds(j * 128, 128)],
                                         acc.at[dbuf.at[j]], ssem, add=True))
                for dsc in puts:
                    dsc.wait()
                return carry

            lax.fori_loop(0, ITERS, _body, 0)
            plsc.subcore_barrier()
            pltpu.sync_copy(acc.at[pl.ds(s * ACC_TILE, ACC_TILE)],
                            out_hbm.at[chunk, pl.ds(s * ACC_TILE, ACC_TILE)])
            plsc.subcore_barrier()


def kernel(x, edge_index, batch, W_h1, b_h1, W_h2, b_h2, W_enc, b_enc,
           ln_g, ln_b, W1, b1, W2, b2, W_dec, b_dec):
    xs = _head(x.reshape(B, IN_DIM), W_h1, b_h1.reshape(1, -1),
               W_h2, b_h2.reshape(1, -1))

    xp = jnp.concatenate([x, jnp.zeros((NP - N, 1), _f32)], axis=0)
    pad_idx = (N + (jnp.arange(EP - E, dtype=jnp.int32) % (NP - N))).astype(jnp.int32)
    src = jnp.concatenate([edge_index[0], pad_idx])
    dst = jnp.concatenate([edge_index[1], pad_idx])
    srcK = (src[None, :] + (jnp.arange(K, dtype=jnp.int32) * NP)[:, None])
    srcK = srcK.reshape(K * SROWS, 128)
    dst2 = dst.reshape(SROWS, 128)
    zrows = jnp.zeros((ACC_TILE, RW), _f32)
    batch_pad = jnp.concatenate(
        [batch, jnp.full((NP - N,), B, dtype=batch.dtype)])
    oh = (batch_pad[:, None] == jnp.arange(B, dtype=batch.dtype)[None, :])
    oh = oh.astype(_f32)

    h = _enc(xp, W_enc.reshape(1, D), b_enc.reshape(1, D))
    for i in range(L):
        t, u = _pre(h, ln_g[i].reshape(1, D), ln_b[i].reshape(1, D))
        sw = _sc_spmm(u.reshape(K * NP, RW), srcK, dst2, zrows)
        h = _post(sw, t, h, W1[i], b1[i].reshape(1, -1),
                  W2[i], b2[i].reshape(1, -1))

    o = _final(h, oh, xs, W_dec, b_dec.reshape(1, 1))
    return o[:, 0]
